# baseline (speedup 1.0000x reference)
import functools

import jax
import jax.numpy as jnp
from jax import lax
from jax.experimental import pallas as pl
from jax.experimental.pallas import tpu as pltpu
from jax.experimental.pallas import tpu_sc as plsc

_VOCAB = 100000
_D = 64
_B = 4096
_CTX = 20

_DP = 128
_NC = 2
_NS = 16
_NW = _NC * _NS
_RW = _B // _NW

_VT = 1536
_NV = (_VOCAB + _VT - 1) // _VT


_CB = 16
_NCH = _RW // _CB
_GSZ = 64
_GPC = (_CB * _CTX) // _GSZ
_GPW = _NCH * _GPC


def _gather_mean_body(xf_hbm, emb_hbm, h_hbm, idx2, rows0, rows1, out_v,
                      sem0, sem1):
    wid = lax.axis_index("s") * _NC + lax.axis_index("c")
    base = wid * _RW

    pltpu.sync_copy(xf_hbm.at[wid], idx2)

    bufs = (rows0, rows1)
    sems = (sem0, sem1)
    pend = [None, None]

    def fire(c):
        buf, sem = bufs[c % 2], sems[c % 2]
        cps = []
        for k in range(_GPC):
            g = c * _GPC + k
            cps.append(
                pltpu.async_copy(emb_hbm.at[idx2.at[g]],
                                 buf.at[pl.ds(k * _GSZ, _GSZ)], sem))
        pend[c % 2] = cps

    scale = jnp.float32(1.0 / _CTX)
    fire(0)
    for c in range(_NCH):
        if c + 1 < _NCH:
            fire(c + 1)
        for cp in pend[c % 2]:
            cp.wait()
        buf = bufs[c % 2]

        def acc_row(r, carry):
            row0 = r * _CTX
            for dd in range(_D // 16):
                sl = pl.ds(dd * 16, 16)
                a = buf[row0, sl]
                for j in range(1, _CTX):
                    a = a + buf[row0 + j, sl]
                out_v[r, sl] = a * scale
            return carry

        lax.fori_loop(0, _CB, acc_row, 0)
        pltpu.sync_copy(out_v, h_hbm.at[pl.ds(base + c * _CB, _CB)])


_gather_mean = functools.partial(
    pl.kernel,
    mesh=plsc.VectorSubcoreMesh(core_axis_name="c", subcore_axis_name="s"),
    out_type=jax.ShapeDtypeStruct((_B, _D), jnp.float32),
    scratch_types=[
        pltpu.VMEM((_GPW, _GSZ), jnp.int32),
        pltpu.VMEM((_CB * _CTX, _DP), jnp.float32),
        pltpu.VMEM((_CB * _CTX, _DP), jnp.float32),
        pltpu.VMEM((_CB, _D), jnp.float32),
        pltpu.SemaphoreType.DMA,
        pltpu.SemaphoreType.DMA,
    ],
)(_gather_mean_body)


def _mm_body(wt_ref, h_ref, o_ref):
    o_ref[...] = lax.dot_general(
        wt_ref[...],
        h_ref[...],
        dimension_numbers=(((0,), (1,)), ((), ())),
        preferred_element_type=jnp.float32,
    )


_matmul_t = pl.pallas_call(
    _mm_body,
    grid=(_NV,),
    in_specs=[
        pl.BlockSpec((_D, _VT), lambda i: (0, i)),
        pl.BlockSpec((_B, _D), lambda i: (0, 0)),
    ],
    out_specs=pl.BlockSpec((_VT, _B), lambda i: (i, 0)),
    out_shape=jax.ShapeDtypeStruct((_VOCAB, _B), jnp.float32),
    compiler_params=pltpu.CompilerParams(vmem_limit_bytes=112 * 1024 * 1024),
)


def kernel(x, emb, W_out):
    xf = x.astype(jnp.int32).reshape(_NW, _GPW, _GSZ)
    emb_p = jnp.pad(emb, ((0, 0), (0, _DP - _D)))
    h = _gather_mean(xf, emb_p)
    logits_t = _matmul_t(W_out.T, h)
    return logits_t.T


# speedup vs baseline: 1.0022x; 1.0022x over previous
import functools

import jax
import jax.numpy as jnp
from jax import lax
from jax.experimental import pallas as pl
from jax.experimental.pallas import tpu as pltpu
from jax.experimental.pallas import tpu_sc as plsc

_VOCAB = 100000
_D = 64
_B = 4096
_CTX = 20

_DP = 128
_NC = 2
_NS = 16
_NW = _NC * _NS
_RW = _B // _NW

_VT = 1024
_NV = (_VOCAB + _VT - 1) // _VT


_CB = 16
_NCH = _RW // _CB
_GSZ = 64
_GPC = (_CB * _CTX) // _GSZ
_GPW = _NCH * _GPC


def _gather_mean_body(xf_hbm, emb_hbm, h_hbm, idx2, rows0, rows1, out_v,
                      sem0, sem1):
    wid = lax.axis_index("s") * _NC + lax.axis_index("c")
    base = wid * _RW

    pltpu.sync_copy(xf_hbm.at[wid], idx2)

    bufs = (rows0, rows1)
    sems = (sem0, sem1)
    pend = [None, None]

    def fire(c):
        buf, sem = bufs[c % 2], sems[c % 2]
        cps = []
        for k in range(_GPC):
            g = c * _GPC + k
            cps.append(
                pltpu.async_copy(emb_hbm.at[idx2.at[g]],
                                 buf.at[pl.ds(k * _GSZ, _GSZ)], sem))
        pend[c % 2] = cps

    scale = jnp.float32(1.0 / _CTX)
    fire(0)
    for c in range(_NCH):
        if c + 1 < _NCH:
            fire(c + 1)
        for cp in pend[c % 2]:
            cp.wait()
        buf = bufs[c % 2]

        def acc_row(r, carry):
            row0 = r * _CTX
            for dd in range(_D // 16):
                sl = pl.ds(dd * 16, 16)
                a = buf[row0, sl]
                for j in range(1, _CTX):
                    a = a + buf[row0 + j, sl]
                out_v[r, sl] = a * scale
            return carry

        lax.fori_loop(0, _CB, acc_row, 0)
        pltpu.sync_copy(out_v, h_hbm.at[pl.ds(base + c * _CB, _CB)])


_gather_mean = functools.partial(
    pl.kernel,
    mesh=plsc.VectorSubcoreMesh(core_axis_name="c", subcore_axis_name="s"),
    out_type=jax.ShapeDtypeStruct((_B, _D), jnp.float32),
    scratch_types=[
        pltpu.VMEM((_GPW, _GSZ), jnp.int32),
        pltpu.VMEM((_CB * _CTX, _DP), jnp.float32),
        pltpu.VMEM((_CB * _CTX, _DP), jnp.float32),
        pltpu.VMEM((_CB, _D), jnp.float32),
        pltpu.SemaphoreType.DMA,
        pltpu.SemaphoreType.DMA,
    ],
)(_gather_mean_body)


def _mm_body(wt_ref, h_ref, o_ref):
    o_ref[...] = lax.dot_general(
        wt_ref[...],
        h_ref[...],
        dimension_numbers=(((0,), (1,)), ((), ())),
        preferred_element_type=jnp.float32,
    )


_matmul_t = pl.pallas_call(
    _mm_body,
    grid=(_NV,),
    in_specs=[
        pl.BlockSpec((_D, _VT), lambda i: (0, i)),
        pl.BlockSpec((_B, _D), lambda i: (0, 0)),
    ],
    out_specs=pl.BlockSpec((_VT, _B), lambda i: (i, 0)),
    out_shape=jax.ShapeDtypeStruct((_VOCAB, _B), jnp.float32),
    compiler_params=pltpu.CompilerParams(vmem_limit_bytes=112 * 1024 * 1024),
)


def kernel(x, emb, W_out):
    xf = x.astype(jnp.int32).reshape(_NW, _GPW, _GSZ)
    emb_p = jnp.pad(emb, ((0, 0), (0, _DP - _D)))
    h = _gather_mean(xf, emb_p)
    logits_t = _matmul_t(W_out.T, h)
    return logits_t.T
